# Initial kernel scaffold; baseline (speedup 1.0000x reference)
import jax
import jax.numpy as jnp
from jax.experimental import pallas as pl


def kernel(x, edge_index, edge_attr, W_self, W_nbr, b):
    raise NotImplementedError("write your pallas kernel here")


# speedup vs baseline: 3.0944x; 3.0944x over previous
import functools

import jax
import jax.numpy as jnp
from jax import lax
from jax.experimental import pallas as pl
from jax.experimental.pallas import tpu as pltpu
from jax.experimental.pallas import tpu_sc as plsc

N = 10000
D = 128
DH = D // 2
E = 320000
NC = 2
NS = 16
C = 128
NCHUNK = 157
EPT = NCHUNK * C
E_PAD = NS * EPT

ZSLAB = 640
ZLAST = N - 15 * ZSLAB


def _sc_aggregate(x2, src3, dst3, attr_rep, zeros):
    mesh = plsc.VectorSubcoreMesh(core_axis_name="c", subcore_axis_name="s")

    @functools.partial(
        pl.kernel,
        out_type=jax.ShapeDtypeStruct((2, N, DH), jnp.float32),
        mesh=mesh,
        compiler_params=pltpu.CompilerParams(use_tc_tiling_on_sc=False),
        scratch_types=[
            pltpu.VMEM((NCHUNK, C), jnp.int32),
            pltpu.VMEM((NCHUNK, C), jnp.int32),
            pltpu.VMEM((C, 16), jnp.float32),
            pltpu.VMEM((C, DH), jnp.float32),
            pltpu.VMEM_SHARED((N, DH), jnp.float32),
            pltpu.SemaphoreType.DMA,
        ],
    )
    def k(x_hbm, src_hbm, dst_hbm, attr_hbm, z_hbm, out_hbm,
          src_v, dst_v, arep_v, rows_v, agg_s, sem):
        c = lax.axis_index("c")
        s = lax.axis_index("s")

        pltpu.sync_copy(src_hbm.at[s], src_v)
        pltpu.sync_copy(dst_hbm.at[s], dst_v)

        @pl.when(s < 15)
        def _():
            pltpu.sync_copy(z_hbm.at[pl.ds(s * ZSLAB, ZSLAB)],
                            agg_s.at[pl.ds(s * ZSLAB, ZSLAB)])

        @pl.when(s == 15)
        def _():
            pltpu.sync_copy(z_hbm.at[pl.ds(15 * ZSLAB, ZLAST)],
                            agg_s.at[pl.ds(15 * ZSLAB, ZLAST)])

        plsc.subcore_barrier()

        def chunk_body(j, carry):
            gather = pltpu.async_copy(x_hbm.at[c].at[src_v.at[j]], rows_v, sem)
            pltpu.sync_copy(attr_hbm.at[s * NCHUNK + j], arep_v)
            gather.wait()

            def edge_body(e, cc):
                a = arep_v[e, :]
                for kk in range(DH // 16):
                    sl = pl.ds(kk * 16, 16)
                    rows_v[e, sl] = rows_v[e, sl] * a
                return cc

            lax.fori_loop(0, C, edge_body, 0)
            pltpu.sync_copy(rows_v, agg_s.at[dst_v.at[j]], add=True)
            return carry

        lax.fori_loop(0, NCHUNK, chunk_body, 0)
        plsc.subcore_barrier()

        @pl.when(s < 15)
        def _():
            pltpu.sync_copy(agg_s.at[pl.ds(s * ZSLAB, ZSLAB)],
                            out_hbm.at[c, pl.ds(s * ZSLAB, ZSLAB)])

        @pl.when(s == 15)
        def _():
            pltpu.sync_copy(agg_s.at[pl.ds(15 * ZSLAB, ZLAST)],
                            out_hbm.at[c, pl.ds(15 * ZSLAB, ZLAST)])

    return k(x2, src3, dst3, attr_rep, zeros)


def _tc_combine(x, agg2, W_self, W_nbr, b2):
    BR = 400
    G = N // BR

    def body(x_ref, a0_ref, a1_ref, ws_ref, wn0_ref, wn1_ref, b_ref, o_ref):
        o_ref[...] = (
            jnp.dot(x_ref[...], ws_ref[...], preferred_element_type=jnp.float32)
            + jnp.dot(a0_ref[0], wn0_ref[...], preferred_element_type=jnp.float32)
            + jnp.dot(a1_ref[0], wn1_ref[...], preferred_element_type=jnp.float32)
            + b_ref[...]
        )

    return pl.pallas_call(
        body,
        grid=(G,),
        in_specs=[
            pl.BlockSpec((BR, D), lambda i: (i, 0)),
            pl.BlockSpec((1, BR, DH), lambda i: (0, i, 0)),
            pl.BlockSpec((1, BR, DH), lambda i: (1, i, 0)),
            pl.BlockSpec((D, D), lambda i: (0, 0)),
            pl.BlockSpec((DH, D), lambda i: (0, 0)),
            pl.BlockSpec((DH, D), lambda i: (1, 0)),
            pl.BlockSpec((1, D), lambda i: (0, 0)),
        ],
        out_specs=pl.BlockSpec((BR, D), lambda i: (i, 0)),
        out_shape=jax.ShapeDtypeStruct((N, D), jnp.float32),
    )(x, agg2, agg2, W_self, W_nbr, W_nbr, b2)


def kernel(x, edge_index, edge_attr, W_self, W_nbr, b):
    src = edge_index[0].astype(jnp.int32)
    dst = edge_index[1].astype(jnp.int32)
    pad = E_PAD - E
    src3 = jnp.pad(src, (0, pad)).reshape(NS, NCHUNK, C)
    dst3 = jnp.pad(dst, (0, pad)).reshape(NS, NCHUNK, C)
    attr_rep = jnp.broadcast_to(
        jnp.pad(edge_attr, (0, pad)).reshape(NS * NCHUNK, C, 1),
        (NS * NCHUNK, C, 16),
    )
    x2 = jnp.stack([x[:, :DH], x[:, DH:]])
    zeros = jnp.zeros((N, DH), jnp.float32)
    agg2 = _sc_aggregate(x2, src3, dst3, attr_rep, zeros)
    return _tc_combine(x, agg2, W_self, W_nbr, b.reshape(1, D))


# speedup vs baseline: 4.1715x; 1.3481x over previous
import functools

import jax
import jax.numpy as jnp
from jax import lax
from jax.experimental import pallas as pl
from jax.experimental.pallas import tpu as pltpu
from jax.experimental.pallas import tpu_sc as plsc

N = 10000
D = 128
DH = D // 2
E = 320000
NC = 2
NS = 16
C = 128
NCHUNK = 158
EPT = NCHUNK * C
E_PAD = NS * EPT

ZSLAB = 640
ZLAST = N - 15 * ZSLAB


def _sc_aggregate(x2, src3, dst3, attr_rep, zeros):
    mesh = plsc.VectorSubcoreMesh(core_axis_name="c", subcore_axis_name="s")

    @functools.partial(
        pl.kernel,
        out_type=jax.ShapeDtypeStruct((2, N, DH), jnp.float32),
        mesh=mesh,
        compiler_params=pltpu.CompilerParams(use_tc_tiling_on_sc=False),
        scratch_types=[
            pltpu.VMEM((NCHUNK, C), jnp.int32),
            pltpu.VMEM((NCHUNK, C), jnp.int32),
            pltpu.VMEM((C, 16), jnp.float32),
            pltpu.VMEM((C, 16), jnp.float32),
            pltpu.VMEM((C, DH), jnp.float32),
            pltpu.VMEM((C, DH), jnp.float32),
            pltpu.VMEM_SHARED((N, DH), jnp.float32),
            pltpu.SemaphoreType.DMA,
            pltpu.SemaphoreType.DMA,
            pltpu.SemaphoreType.DMA,
            pltpu.SemaphoreType.DMA,
        ],
    )
    def k(x_hbm, src_hbm, dst_hbm, attr_hbm, z_hbm, out_hbm,
          src_v, dst_v, arep0, arep1, rows0, rows1, agg_s,
          g0sem, g1sem, s0sem, s1sem):
        c = lax.axis_index("c")
        s = lax.axis_index("s")

        pltpu.sync_copy(src_hbm.at[s], src_v)
        pltpu.sync_copy(dst_hbm.at[s], dst_v)

        @pl.when(s < 15)
        def _():
            pltpu.sync_copy(z_hbm.at[pl.ds(s * ZSLAB, ZSLAB)],
                            agg_s.at[pl.ds(s * ZSLAB, ZSLAB)])

        @pl.when(s == 15)
        def _():
            pltpu.sync_copy(z_hbm.at[pl.ds(15 * ZSLAB, ZLAST)],
                            agg_s.at[pl.ds(15 * ZSLAB, ZLAST)])

        plsc.subcore_barrier()

        def issue_gather(j, rows_b, arep_b, gsem):
            pltpu.async_copy(x_hbm.at[c].at[src_v.at[j]], rows_b, gsem)
            pltpu.async_copy(attr_hbm.at[s * NCHUNK + j], arep_b, gsem)

        def wait_gather(j, rows_b, arep_b, gsem):
            pltpu.make_async_copy(x_hbm.at[c].at[src_v.at[j]], rows_b,
                                  gsem).wait()
            pltpu.make_async_copy(attr_hbm.at[s * NCHUNK + j], arep_b,
                                  gsem).wait()

        def scale(rows_b, arep_b):
            @plsc.parallel_loop(0, C, unroll=8)
            def _(e):
                a = arep_b[e, :]
                for kk in range(DH // 16):
                    sl = pl.ds(kk * 16, 16)
                    rows_b[e, sl] = rows_b[e, sl] * a

        issue_gather(0, rows0, arep0, g0sem)
        issue_gather(1, rows1, arep1, g1sem)

        def pair_body(p, carry):
            j0 = 2 * p
            j1 = j0 + 1

            wait_gather(j0, rows0, arep0, g0sem)
            scale(rows0, arep0)
            pltpu.async_copy(rows0, agg_s.at[dst_v.at[j0]], s0sem, add=True)

            wait_gather(j1, rows1, arep1, g1sem)
            scale(rows1, arep1)
            pltpu.async_copy(rows1, agg_s.at[dst_v.at[j1]], s1sem, add=True)

            @pl.when(p < NCHUNK // 2 - 1)
            def _():
                pltpu.make_async_copy(rows0, agg_s.at[dst_v.at[j0]],
                                      s0sem).wait()
                issue_gather(j0 + 2, rows0, arep0, g0sem)
                pltpu.make_async_copy(rows1, agg_s.at[dst_v.at[j1]],
                                      s1sem).wait()
                issue_gather(j1 + 2, rows1, arep1, g1sem)

            @pl.when(p == NCHUNK // 2 - 1)
            def _():
                pltpu.make_async_copy(rows0, agg_s.at[dst_v.at[j0]],
                                      s0sem).wait()
                pltpu.make_async_copy(rows1, agg_s.at[dst_v.at[j1]],
                                      s1sem).wait()

            return carry

        lax.fori_loop(0, NCHUNK // 2, pair_body, 0)
        plsc.subcore_barrier()

        @pl.when(s < 15)
        def _():
            pltpu.sync_copy(agg_s.at[pl.ds(s * ZSLAB, ZSLAB)],
                            out_hbm.at[c, pl.ds(s * ZSLAB, ZSLAB)])

        @pl.when(s == 15)
        def _():
            pltpu.sync_copy(agg_s.at[pl.ds(15 * ZSLAB, ZLAST)],
                            out_hbm.at[c, pl.ds(15 * ZSLAB, ZLAST)])

    return k(x2, src3, dst3, attr_rep, zeros)


def _tc_combine(x, agg2, W_self, W_nbr, b2):
    BR = 400
    G = N // BR

    def body(x_ref, a0_ref, a1_ref, ws_ref, wn0_ref, wn1_ref, b_ref, o_ref):
        o_ref[...] = (
            jnp.dot(x_ref[...], ws_ref[...], preferred_element_type=jnp.float32)
            + jnp.dot(a0_ref[0], wn0_ref[...], preferred_element_type=jnp.float32)
            + jnp.dot(a1_ref[0], wn1_ref[...], preferred_element_type=jnp.float32)
            + b_ref[...]
        )

    return pl.pallas_call(
        body,
        grid=(G,),
        in_specs=[
            pl.BlockSpec((BR, D), lambda i: (i, 0)),
            pl.BlockSpec((1, BR, DH), lambda i: (0, i, 0)),
            pl.BlockSpec((1, BR, DH), lambda i: (1, i, 0)),
            pl.BlockSpec((D, D), lambda i: (0, 0)),
            pl.BlockSpec((DH, D), lambda i: (0, 0)),
            pl.BlockSpec((DH, D), lambda i: (1, 0)),
            pl.BlockSpec((1, D), lambda i: (0, 0)),
        ],
        out_specs=pl.BlockSpec((BR, D), lambda i: (i, 0)),
        out_shape=jax.ShapeDtypeStruct((N, D), jnp.float32),
    )(x, agg2, agg2, W_self, W_nbr, W_nbr, b2)


def kernel(x, edge_index, edge_attr, W_self, W_nbr, b):
    src = edge_index[0].astype(jnp.int32)
    dst = edge_index[1].astype(jnp.int32)
    pad = E_PAD - E
    src3 = jnp.pad(src, (0, pad)).reshape(NS, NCHUNK, C)
    dst3 = jnp.pad(dst, (0, pad)).reshape(NS, NCHUNK, C)
    attr_rep = jnp.broadcast_to(
        jnp.pad(edge_attr, (0, pad)).reshape(NS * NCHUNK, C, 1),
        (NS * NCHUNK, C, 16),
    )
    x2 = jnp.stack([x[:, :DH], x[:, DH:]])
    zeros = jnp.zeros((N, DH), jnp.float32)
    agg2 = _sc_aggregate(x2, src3, dst3, attr_rep, zeros)
    return _tc_combine(x, agg2, W_self, W_nbr, b.reshape(1, D))
